# Initial kernel scaffold; baseline (speedup 1.0000x reference)
import jax
import jax.numpy as jnp
from jax.experimental import pallas as pl


def kernel(input, mlp1_w0, mlp1_b0, mlp1_w1, mlp1_b1, mlp1_w2, mlp1_b2, lin1_w, lin1_b, conv1_w, conv1_b, lin2_w, lin2_b, conv2_w, conv2_b, mlp2_w0, mlp2_b0, mlp2_w1, mlp2_b1, f1_w0, f1_b0, f1_w1, f1_b1, f1_w2, f1_b2, f2_w0, f2_b0, f2_w1, f2_b1, f2_w2, f2_b2):
    raise NotImplementedError("write your pallas kernel here")


# speedup vs baseline: 3.3826x; 3.3826x over previous
import functools

import numpy as np
import jax
import jax.numpy as jnp
from jax import lax
from jax.experimental import pallas as pl
from jax.experimental.pallas import tpu as pltpu

_M = 2048
_K = 16
_B = 8
_N = 2048
_TQ = 256
_TD = 512


def _make_sphere():
    phi = np.pi * (3.0 - np.sqrt(5.0))
    i = np.arange(_M, dtype=np.float64)
    y = 1.0 - i / float(_M - 1) * 2.0
    radius = np.sqrt(np.clip(1.0 - y * y, 0.0, None))
    theta = phi * i
    x = np.cos(theta) * radius
    z = np.sin(theta) * radius
    return np.stack([x, y, z], axis=1)


_SPHERE = jnp.asarray(_make_sphere() * 100.0, dtype=jnp.float32)


def _dot_t(a, b):
    return lax.dot_general(a, b, (((1,), (1,)), ((), ())),
                           preferred_element_type=jnp.float32)


def _knn_body(xq_ref, xa_ref, w0_ref, b0_ref, w1_ref, b1_ref, w2_ref, b2_ref,
              idx_ref, feat_ref):
    b = pl.program_id(0)
    xq = xq_ref[0]
    xa = xa_ref[0]
    inner = -2.0 * _dot_t(xq, xa)
    xxq = jnp.sum(xq * xq, axis=1)
    xxa = jnp.sum(xa * xa, axis=1)
    pd = -xxq[:, None] - inner - xxa[None, :]
    jidx = lax.broadcasted_iota(jnp.int32, pd.shape, 1)
    cols = []
    g01 = []
    for k in range(_K):
        m = jnp.max(pd, axis=1)
        cand = jnp.where(pd == m[:, None], jidx, _N)
        a = jnp.min(cand, axis=1)
        cols.append(a)
        hit = jidx == a[:, None]
        if k < 2:
            g01.append(lax.dot_general(hit.astype(jnp.float32), xa,
                                       (((1,), (0,)), ((), ())),
                                       preferred_element_type=jnp.float32))
        if k < _K - 1:
            pd = jnp.where(hit, -jnp.inf, pd)
    idx_ref[0] = (jnp.concatenate([c[:, None] for c in cols], axis=1)
                  + b * _N)
    g0, g1 = g01
    cov = jnp.concatenate([xq] + [g0[:, a:a + 1] * g1 for a in range(3)],
                          axis=1)
    h = jnp.maximum(_dot_t(cov, w0_ref[...]) + b0_ref[...], 0.0)
    h = jnp.maximum(_dot_t(h, w1_ref[...]) + b1_ref[...], 0.0)
    h = jnp.maximum(_dot_t(h, w2_ref[...]) + b2_ref[...], 0.0)
    feat_ref[0] = h


def _knn_mlp1(inp, w0, b0, w1, b1, w2, b2):
    grid = (_B, _N // _TQ)
    full = lambda shape: pl.BlockSpec(shape, lambda b, t: (0, 0))
    idx, feat = pl.pallas_call(
        _knn_body,
        grid=grid,
        in_specs=[
            pl.BlockSpec((1, _TQ, 3), lambda b, t: (b, t, 0)),
            pl.BlockSpec((1, _N, 3), lambda b, t: (b, 0, 0)),
            full((64, 12)), full((1, 64)),
            full((64, 64)), full((1, 64)),
            full((64, 64)), full((1, 64)),
        ],
        out_specs=[
            pl.BlockSpec((1, _TQ, _K), lambda b, t: (b, t, 0)),
            pl.BlockSpec((1, _TQ, 64), lambda b, t: (b, t, 0)),
        ],
        out_shape=[
            jax.ShapeDtypeStruct((_B, _N, _K), jnp.int32),
            jax.ShapeDtypeStruct((_B, _N, 64), jnp.float32),
        ],
    )(inp, inp, w0, b0.reshape(1, -1), w1, b1.reshape(1, -1),
      w2, b2.reshape(1, -1))
    return idx, feat


def _gather_max(x_flat, gidx):
    g = x_flat[gidx.reshape(-1)].reshape(_B * _N, _K, -1)
    return jnp.max(g, axis=1)


def _g1_body(x_ref, lw_ref, lb_ref, cw_ref, cb_ref, o_ref):
    y = _dot_t(x_ref[...], lw_ref[...]) + lb_ref[...]
    o_ref[...] = jnp.maximum(_dot_t(y, cw_ref[...]) + cb_ref[...], 0.0)


def _graph1(x, lw, lb, cw, cb):
    full = lambda shape: pl.BlockSpec(shape, lambda i: (0, 0))
    return pl.pallas_call(
        _g1_body,
        grid=(_B * _N // _TD,),
        in_specs=[pl.BlockSpec((_TD, 64), lambda i: (i, 0)),
                  full((64, 64)), full((1, 64)),
                  full((128, 64)), full((1, 128))],
        out_specs=pl.BlockSpec((_TD, 128), lambda i: (i, 0)),
        out_shape=jax.ShapeDtypeStruct((_B * _N, 128), jnp.float32),
    )(x, lw, lb.reshape(1, -1), cw, cb.reshape(1, -1))


def _g2_body(x_ref, lw_ref, lb_ref, cw_ref, cb_ref, o_ref):
    t = pl.program_id(1)
    y = _dot_t(x_ref[0], lw_ref[...]) + lb_ref[...]
    z = _dot_t(y, cw_ref[...]) + cb_ref[...]
    m = jnp.max(z, axis=0)[None, None, :]

    @pl.when(t == 0)
    def _():
        o_ref[...] = jnp.full_like(o_ref, -jnp.inf)
    o_ref[...] = jnp.maximum(o_ref[...], m)


def _graph2(x, lw, lb, cw, cb):
    full = lambda shape: pl.BlockSpec(shape, lambda b, t: (0, 0))
    return pl.pallas_call(
        _g2_body,
        grid=(_B, _N // _TD),
        in_specs=[pl.BlockSpec((1, _TD, 128), lambda b, t: (b, t, 0)),
                  full((128, 128)), full((1, 128)),
                  full((1024, 128)), full((1, 1024))],
        out_specs=pl.BlockSpec((1, 1, 1024), lambda b, t: (b, 0, 0)),
        out_shape=jax.ShapeDtypeStruct((_B, 1, 1024), jnp.float32),
    )(x.reshape(_B, _N, 128), lw, lb.reshape(1, -1), cw,
      cb.reshape(1, -1)).reshape(_B, 1024)


def _head_body(x_ref, w0_ref, b0_ref, w1_ref, b1_ref, o_ref):
    h = jnp.maximum(_dot_t(x_ref[...], w0_ref[...]) + b0_ref[...], 0.0)
    o_ref[...] = _dot_t(h, w1_ref[...]) + b1_ref[...]


def _head(x, w0, b0, w1, b1):
    full = lambda shape: pl.BlockSpec(shape, lambda: (0, 0))
    return pl.pallas_call(
        _head_body,
        in_specs=[full((_B, 1024)), full((512, 1024)), full((1, 512)),
                  full((512, 512)), full((1, 512))],
        out_specs=full((_B, 512)),
        out_shape=jax.ShapeDtypeStruct((_B, 512), jnp.float32),
    )(x, w0, b0.reshape(1, -1), w1, b1.reshape(1, -1))


def _dec_body(f_ref, sp_ref, w0a_ref, w0b_ref, b0_ref, w1_ref, b1_ref,
              w2_ref, b2_ref, v0a_ref, v0b_ref, c0_ref, v1_ref, c1_ref,
              v2_ref, c2_ref, o_ref):
    f = f_ref[0]
    sp = sp_ref[...]
    c1 = _dot_t(f, w0a_ref[...]) + b0_ref[...]
    h = jnp.maximum(c1 + _dot_t(sp, w0b_ref[...]), 0.0)
    h = jnp.maximum(_dot_t(h, w1_ref[...]) + b1_ref[...], 0.0)
    f1 = _dot_t(h, w2_ref[...]) + b2_ref[...]
    c2 = _dot_t(f, v0a_ref[...]) + c0_ref[...]
    h2 = jnp.maximum(c2 + _dot_t(f1, v0b_ref[...]), 0.0)
    h2 = jnp.maximum(_dot_t(h2, v1_ref[...]) + c1_ref[...], 0.0)
    o_ref[0] = _dot_t(h2, v2_ref[...]) + c2_ref[...]


def _decoder(feat, f1_w0, f1_b0, f1_w1, f1_b1, f1_w2, f1_b2,
             f2_w0, f2_b0, f2_w1, f2_b1, f2_w2, f2_b2):
    full = lambda shape: pl.BlockSpec(shape, lambda b, t: (0, 0))
    return pl.pallas_call(
        _dec_body,
        grid=(_B, _M // _TD),
        in_specs=[pl.BlockSpec((1, 1, 512), lambda b, t: (b, 0, 0)),
                  pl.BlockSpec((_TD, 3), lambda b, t: (t, 0)),
                  full((512, 512)), full((512, 3)), full((1, 512)),
                  full((512, 512)), full((1, 512)),
                  full((3, 512)), full((1, 3)),
                  full((512, 512)), full((512, 3)), full((1, 512)),
                  full((512, 512)), full((1, 512)),
                  full((3, 512)), full((1, 3))],
        out_specs=pl.BlockSpec((1, _TD, 3), lambda b, t: (b, t, 0)),
        out_shape=jax.ShapeDtypeStruct((_B, _M, 3), jnp.float32),
    )(feat.reshape(_B, 1, 512), _SPHERE,
      f1_w0[:, :512], f1_w0[:, 512:], f1_b0.reshape(1, -1),
      f1_w1, f1_b1.reshape(1, -1), f1_w2, f1_b2.reshape(1, -1),
      f2_w0[:, :512], f2_w0[:, 512:], f2_b0.reshape(1, -1),
      f2_w1, f2_b1.reshape(1, -1), f2_w2, f2_b2.reshape(1, -1))


def kernel(input, mlp1_w0, mlp1_b0, mlp1_w1, mlp1_b1, mlp1_w2, mlp1_b2,
           lin1_w, lin1_b, conv1_w, conv1_b, lin2_w, lin2_b, conv2_w,
           conv2_b, mlp2_w0, mlp2_b0, mlp2_w1, mlp2_b1, f1_w0, f1_b0,
           f1_w1, f1_b1, f1_w2, f1_b2, f2_w0, f2_b0, f2_w1, f2_b1,
           f2_w2, f2_b2):
    idx, feat1 = _knn_mlp1(input, mlp1_w0, mlp1_b0, mlp1_w1, mlp1_b1,
                           mlp1_w2, mlp1_b2)
    pooled1 = _gather_max(feat1.reshape(_B * _N, 64), idx)
    feat2 = _graph1(pooled1, lin1_w, lin1_b, conv1_w, conv1_b)
    pooled2 = _gather_max(feat2, idx)
    gmax = _graph2(pooled2, lin2_w, lin2_b, conv2_w, conv2_b)
    featv = _head(gmax, mlp2_w0, mlp2_b0, mlp2_w1, mlp2_b1)
    return _decoder(featv, f1_w0, f1_b0, f1_w1, f1_b1, f1_w2, f1_b2,
                    f2_w0, f2_b0, f2_w1, f2_b1, f2_w2, f2_b2)
